# Initial kernel scaffold; baseline (speedup 1.0000x reference)
import jax
import jax.numpy as jnp
from jax.experimental import pallas as pl


def kernel(filtres, X):
    raise NotImplementedError("write your pallas kernel here")


# speedup vs baseline: 4.0690x; 4.0690x over previous
import functools

import jax
import jax.numpy as jnp
from jax import lax
from jax.experimental import pallas as pl
from jax.experimental.pallas import tpu as pltpu
from jax.experimental.pallas import tpu_sc as plsc

BATCH = 16
N_NODES = 4096
F = 128
NVEC = F // 16
ROWS_PER_W = N_NODES // 2
RCHUNK = 256
NCHUNK = ROWS_PER_W // RCHUNK
UNROLL = 4

_mesh = plsc.VectorSubcoreMesh(core_axis_name="c", subcore_axis_name="s")


@functools.partial(
    pl.kernel,
    out_type=jax.ShapeDtypeStruct((BATCH * F,), jnp.float32),
    mesh=_mesh,
    scratch_types=[
        pltpu.VMEM((RCHUNK * F,), jnp.float32),
        pltpu.VMEM((RCHUNK * F,), jnp.float32),
        pltpu.VMEM((F,), jnp.float32),
        pltpu.VMEM((F,), jnp.float32),
        pltpu.VMEM_SHARED((16 * F,), jnp.float32),
        pltpu.SemaphoreType.DMA,
        pltpu.SemaphoreType.DMA,
    ],
)
def _segment_max_sc(x_hbm, out_hbm, buf0, buf1, res, res2, shared, sem0, sem1):
    c = lax.axis_index("c")
    s = lax.axis_index("s")
    b = c * 8 + s // 2
    h = s % 2
    base = (b * N_NODES + h * ROWS_PER_W) * F

    bufs = (buf0, buf1)
    sems = (sem0, sem1)

    def start(i):
        return pltpu.async_copy(
            x_hbm.at[pl.ds(base + i * RCHUNK * F, RCHUNK * F)],
            bufs[i % 2],
            sems[i % 2],
        )

    def fold_chunk(buf, acc):
        def body(it, acc):
            for k in range(UNROLL):
                r = it * UNROLL + k
                acc = tuple(
                    jnp.maximum(acc[j], buf[pl.ds(r * F + j * 16, 16)])
                    for j in range(NVEC)
                )
            return acc
        return lax.fori_loop(0, RCHUNK // UNROLL, body, acc)

    acc = tuple(jnp.full((16,), -jnp.inf, jnp.float32) for _ in range(NVEC))
    pending = start(0)
    for i in range(NCHUNK):
        nxt = start(i + 1) if i + 1 < NCHUNK else None
        pending.wait()
        acc = fold_chunk(bufs[i % 2], acc)
        pending = nxt

    for j in range(NVEC):
        res[pl.ds(j * 16, 16)] = acc[j]
    pltpu.sync_copy(res, shared.at[pl.ds(s * F, F)])
    plsc.subcore_barrier()

    @pl.when(h == 0)
    def _():
        pltpu.sync_copy(shared.at[pl.ds((s + 1) * F, F)], res2)
        for j in range(NVEC):
            res[pl.ds(j * 16, 16)] = jnp.maximum(
                res[pl.ds(j * 16, 16)], res2[pl.ds(j * 16, 16)]
            )
        pltpu.sync_copy(res, out_hbm.at[pl.ds(b * F, F)])


def kernel(filtres, X):
    pooled = _segment_max_sc(X.reshape(-1))
    return (filtres, pooled.reshape(BATCH, F))


# speedup vs baseline: 4.1093x; 1.0099x over previous
import functools

import jax
import jax.numpy as jnp
from jax import lax
from jax.experimental import pallas as pl
from jax.experimental.pallas import tpu as pltpu
from jax.experimental.pallas import tpu_sc as plsc

BATCH = 16
N_NODES = 4096
F = 128
NVEC = F // 16
ROWS_PER_W = N_NODES // 2
RCHUNK = 256
NCHUNK = ROWS_PER_W // RCHUNK
UNROLL = 4

_mesh = plsc.VectorSubcoreMesh(core_axis_name="c", subcore_axis_name="s")


@functools.partial(
    pl.kernel,
    out_type=jax.ShapeDtypeStruct((BATCH * F,), jnp.float32),
    mesh=_mesh,
    scratch_types=[
        pltpu.VMEM((RCHUNK * F,), jnp.float32),
        pltpu.VMEM((RCHUNK * F,), jnp.float32),
        pltpu.VMEM((F,), jnp.float32),
        pltpu.VMEM((F,), jnp.float32),
        pltpu.VMEM_SHARED((16 * F,), jnp.float32),
        pltpu.SemaphoreType.DMA,
        pltpu.SemaphoreType.DMA,
    ],
)
def _segment_max_sc(x_hbm, out_hbm, buf0, buf1, res, res2, shared, sem0, sem1):
    c = lax.axis_index("c")
    s = lax.axis_index("s")
    b = c * 8 + s // 2
    h = s % 2
    base = (b * N_NODES + h * ROWS_PER_W) * F

    bufs = (buf0, buf1)
    sems = (sem0, sem1)

    def start(i, bsel):
        pltpu.async_copy(
            x_hbm.at[pl.ds(base + i * RCHUNK * F, RCHUNK * F)],
            bufs[bsel],
            sems[bsel],
        )

    def wait(bsel):
        pltpu.make_async_copy(
            x_hbm.at[pl.ds(base, RCHUNK * F)], bufs[bsel], sems[bsel]
        ).wait()

    def fold_chunk(buf, acc):
        def body(it, acc):
            for k in range(UNROLL):
                r = it * UNROLL + k
                acc = tuple(
                    jnp.maximum(acc[j], buf[pl.ds(r * F + j * 16, 16)])
                    for j in range(NVEC)
                )
            return acc
        return lax.fori_loop(0, RCHUNK // UNROLL, body, acc)

    acc = tuple(jnp.full((16,), -jnp.inf, jnp.float32) for _ in range(NVEC))
    start(0, 0)
    start(1, 1)

    def pair_body(i, acc):
        wait(0)
        acc = fold_chunk(buf0, acc)
        start(2 * i + 2, 0)
        wait(1)
        acc = fold_chunk(buf1, acc)
        start(2 * i + 3, 1)
        return acc

    acc = lax.fori_loop(0, NCHUNK // 2 - 1, pair_body, acc)
    wait(0)
    acc = fold_chunk(buf0, acc)
    wait(1)
    acc = fold_chunk(buf1, acc)

    for j in range(NVEC):
        res[pl.ds(j * 16, 16)] = acc[j]
    pltpu.sync_copy(res, shared.at[pl.ds(s * F, F)])
    plsc.subcore_barrier()

    @pl.when(h == 0)
    def _():
        pltpu.sync_copy(shared.at[pl.ds((s + 1) * F, F)], res2)
        for j in range(NVEC):
            res[pl.ds(j * 16, 16)] = jnp.maximum(
                res[pl.ds(j * 16, 16)], res2[pl.ds(j * 16, 16)]
            )
        pltpu.sync_copy(res, out_hbm.at[pl.ds(b * F, F)])


def kernel(filtres, X):
    pooled = _segment_max_sc(X.reshape(-1))
    return (filtres, pooled.reshape(BATCH, F))


# speedup vs baseline: 4.7047x; 1.1449x over previous
import functools

import jax
import jax.numpy as jnp
from jax import lax
from jax.experimental import pallas as pl
from jax.experimental.pallas import tpu as pltpu
from jax.experimental.pallas import tpu_sc as plsc

BATCH = 16
N_NODES = 4096
F = 128
NVEC = F // 16
ROWS_PER_W = N_NODES // 2
RCHUNK = 256
NCHUNK = ROWS_PER_W // RCHUNK
UNROLL = 4

_mesh = plsc.VectorSubcoreMesh(core_axis_name="c", subcore_axis_name="s")


@functools.partial(
    pl.kernel,
    out_type=jax.ShapeDtypeStruct((BATCH * F,), jnp.float32),
    mesh=_mesh,
    scratch_types=[
        pltpu.VMEM((RCHUNK * F,), jnp.float32),
        pltpu.VMEM((RCHUNK * F,), jnp.float32),
        pltpu.VMEM((F,), jnp.float32),
        pltpu.VMEM((F,), jnp.float32),
        pltpu.VMEM_SHARED((16 * F,), jnp.float32),
        pltpu.SemaphoreType.DMA,
        pltpu.SemaphoreType.DMA,
    ],
)
def _segment_max_sc(x_hbm, out_hbm, buf0, buf1, res, res2, shared, sem0, sem1):
    c = lax.axis_index("c")
    s = lax.axis_index("s")
    b = c * 8 + s // 2
    h = s % 2
    base = (b * N_NODES + h * ROWS_PER_W) * F

    bufs = (buf0, buf1)
    sems = (sem0, sem1)

    def start(i, bsel):
        pltpu.async_copy(
            x_hbm.at[pl.ds(base + i * RCHUNK * F, RCHUNK * F)],
            bufs[bsel],
            sems[bsel],
        )

    def wait(bsel):
        pltpu.make_async_copy(
            x_hbm.at[pl.ds(base, RCHUNK * F)], bufs[bsel], sems[bsel]
        ).wait()

    def fold_chunk(buf, acc):
        def body(it, acc):
            for k in range(UNROLL):
                r = it * UNROLL + k
                acc = tuple(
                    jnp.maximum(acc[j], buf[pl.ds(r * F + j * 16, 16)])
                    for j in range(NVEC)
                )
            return acc
        return lax.fori_loop(0, RCHUNK // UNROLL, body, acc)

    acc = tuple(jnp.full((16,), -jnp.inf, jnp.float32) for _ in range(NVEC))
    start(0, 0)
    start(1, 1)

    def pair_body(i, acc):
        wait(0)
        acc = fold_chunk(buf0, acc)
        start(2 * i + 2, 0)
        wait(1)
        acc = fold_chunk(buf1, acc)
        start(2 * i + 3, 1)
        return acc

    acc = lax.fori_loop(0, NCHUNK // 2 - 1, pair_body, acc)
    wait(0)
    acc = fold_chunk(buf0, acc)
    wait(1)
    acc = fold_chunk(buf1, acc)

    for j in range(NVEC):
        res[pl.ds(j * 16, 16)] = acc[j]
    pltpu.sync_copy(res, shared.at[pl.ds(s * F, F)])
    plsc.subcore_barrier()

    @pl.when(h == 0)
    def _():
        pltpu.sync_copy(shared.at[pl.ds((s + 1) * F, F)], res2)
        for j in range(NVEC):
            res[pl.ds(j * 16, 16)] = jnp.maximum(
                res[pl.ds(j * 16, 16)], res2[pl.ds(j * 16, 16)]
            )
        pltpu.sync_copy(res, out_hbm.at[pl.ds(b * F, F)])


def _copy_body(x_ref, o_ref):
    o_ref[...] = x_ref[...]


_filtres_copy = pl.pallas_call(
    _copy_body,
    grid=(BATCH,),
    in_specs=[pl.BlockSpec((1, N_NODES, F), lambda i: (i, 0, 0))],
    out_specs=pl.BlockSpec((1, N_NODES, F), lambda i: (i, 0, 0)),
    out_shape=jax.ShapeDtypeStruct((BATCH, N_NODES, F), jnp.float32),
)


def kernel(filtres, X):
    pooled = _segment_max_sc(X.reshape(-1))
    return (_filtres_copy(filtres), pooled.reshape(BATCH, F))


# speedup vs baseline: 4.7466x; 1.0089x over previous
import functools

import jax
import jax.numpy as jnp
from jax import lax
from jax.experimental import pallas as pl
from jax.experimental.pallas import tpu as pltpu
from jax.experimental.pallas import tpu_sc as plsc

BATCH = 16
N_NODES = 4096
F = 128
NVEC = F // 16
ROWS_PER_W = N_NODES // 2
RCHUNK = 256
NCHUNK = ROWS_PER_W // RCHUNK
UNROLL = 2

_mesh = plsc.VectorSubcoreMesh(core_axis_name="c", subcore_axis_name="s")


@functools.partial(
    pl.kernel,
    out_type=jax.ShapeDtypeStruct((BATCH * F,), jnp.float32),
    mesh=_mesh,
    scratch_types=[
        pltpu.VMEM((RCHUNK * F,), jnp.float32),
        pltpu.VMEM((RCHUNK * F,), jnp.float32),
        pltpu.VMEM((F,), jnp.float32),
        pltpu.VMEM((F,), jnp.float32),
        pltpu.VMEM_SHARED((16 * F,), jnp.float32),
        pltpu.SemaphoreType.DMA,
        pltpu.SemaphoreType.DMA,
    ],
)
def _segment_max_sc(x_hbm, out_hbm, buf0, buf1, res, res2, shared, sem0, sem1):
    c = lax.axis_index("c")
    s = lax.axis_index("s")
    b = c * 8 + s // 2
    h = s % 2
    base = (b * N_NODES + h * ROWS_PER_W) * F

    bufs = (buf0, buf1)
    sems = (sem0, sem1)

    def start(i, bsel):
        pltpu.async_copy(
            x_hbm.at[pl.ds(base + i * RCHUNK * F, RCHUNK * F)],
            bufs[bsel],
            sems[bsel],
        )

    def wait(bsel):
        pltpu.make_async_copy(
            x_hbm.at[pl.ds(base, RCHUNK * F)], bufs[bsel], sems[bsel]
        ).wait()

    def fold_chunk(buf, acc):
        def body(it, acc):
            for k in range(UNROLL):
                r = it * UNROLL + k
                acc = tuple(
                    jnp.maximum(acc[j], buf[pl.ds(r * F + j * 16, 16)])
                    for j in range(NVEC)
                )
            return acc
        return lax.fori_loop(0, RCHUNK // UNROLL, body, acc)

    acc = tuple(jnp.full((16,), -jnp.inf, jnp.float32) for _ in range(NVEC))
    start(0, 0)
    start(1, 1)

    def pair_body(i, acc):
        wait(0)
        acc = fold_chunk(buf0, acc)
        start(2 * i + 2, 0)
        wait(1)
        acc = fold_chunk(buf1, acc)
        start(2 * i + 3, 1)
        return acc

    acc = lax.fori_loop(0, NCHUNK // 2 - 1, pair_body, acc)
    wait(0)
    acc = fold_chunk(buf0, acc)
    wait(1)
    acc = fold_chunk(buf1, acc)

    for j in range(NVEC):
        res[pl.ds(j * 16, 16)] = acc[j]
    pltpu.sync_copy(res, shared.at[pl.ds(s * F, F)])
    plsc.subcore_barrier()

    @pl.when(h == 0)
    def _():
        pltpu.sync_copy(shared.at[pl.ds((s + 1) * F, F)], res2)
        for j in range(NVEC):
            res[pl.ds(j * 16, 16)] = jnp.maximum(
                res[pl.ds(j * 16, 16)], res2[pl.ds(j * 16, 16)]
            )
        pltpu.sync_copy(res, out_hbm.at[pl.ds(b * F, F)])


def _copy_body(x_ref, o_ref):
    o_ref[...] = x_ref[...]


_filtres_copy = pl.pallas_call(
    _copy_body,
    grid=(BATCH,),
    in_specs=[pl.BlockSpec((1, N_NODES, F), lambda i: (i, 0, 0))],
    out_specs=pl.BlockSpec((1, N_NODES, F), lambda i: (i, 0, 0)),
    out_shape=jax.ShapeDtypeStruct((BATCH, N_NODES, F), jnp.float32),
)


def kernel(filtres, X):
    pooled = _segment_max_sc(X.reshape(-1))
    return (_filtres_copy(filtres), pooled.reshape(BATCH, F))


# speedup vs baseline: 4.7617x; 1.0032x over previous
import functools

import jax
import jax.numpy as jnp
from jax import lax
from jax.experimental import pallas as pl
from jax.experimental.pallas import tpu as pltpu
from jax.experimental.pallas import tpu_sc as plsc

BATCH = 16
N_NODES = 4096
F = 128
NVEC = F // 16
TC_ROWS = 256
SC_ROWS = N_NODES - TC_ROWS
ROWS_PER_W = SC_ROWS // 2
RCHUNK = 240
NCHUNK = ROWS_PER_W // RCHUNK
UNROLL = 2

_mesh = plsc.VectorSubcoreMesh(core_axis_name="c", subcore_axis_name="s")


@functools.partial(
    pl.kernel,
    out_type=jax.ShapeDtypeStruct((BATCH * F,), jnp.float32),
    mesh=_mesh,
    scratch_types=[
        pltpu.VMEM((RCHUNK * F,), jnp.float32),
        pltpu.VMEM((RCHUNK * F,), jnp.float32),
        pltpu.VMEM((F,), jnp.float32),
        pltpu.VMEM((F,), jnp.float32),
        pltpu.VMEM_SHARED((16 * F,), jnp.float32),
        pltpu.SemaphoreType.DMA,
        pltpu.SemaphoreType.DMA,
    ],
)
def _segment_max_sc(x_hbm, out_hbm, buf0, buf1, res, res2, shared, sem0, sem1):
    c = lax.axis_index("c")
    s = lax.axis_index("s")
    b = c * 8 + s // 2
    h = s % 2
    base = (b * N_NODES + TC_ROWS + h * ROWS_PER_W) * F

    bufs = (buf0, buf1)
    sems = (sem0, sem1)

    def start(i, bsel):
        pltpu.async_copy(
            x_hbm.at[pl.ds(base + i * RCHUNK * F, RCHUNK * F)],
            bufs[bsel],
            sems[bsel],
        )

    def wait(bsel):
        pltpu.make_async_copy(
            x_hbm.at[pl.ds(base, RCHUNK * F)], bufs[bsel], sems[bsel]
        ).wait()

    def fold_chunk(buf, acc):
        def body(it, acc):
            for k in range(UNROLL):
                r = it * UNROLL + k
                acc = tuple(
                    jnp.maximum(acc[j], buf[pl.ds(r * F + j * 16, 16)])
                    for j in range(NVEC)
                )
            return acc
        return lax.fori_loop(0, RCHUNK // UNROLL, body, acc)

    acc = tuple(jnp.full((16,), -jnp.inf, jnp.float32) for _ in range(NVEC))
    start(0, 0)
    start(1, 1)

    def pair_body(i, acc):
        wait(0)
        acc = fold_chunk(buf0, acc)
        start(2 * i + 2, 0)
        wait(1)
        acc = fold_chunk(buf1, acc)
        start(2 * i + 3, 1)
        return acc

    acc = lax.fori_loop(0, NCHUNK // 2 - 1, pair_body, acc)
    wait(0)
    acc = fold_chunk(buf0, acc)
    wait(1)
    acc = fold_chunk(buf1, acc)

    for j in range(NVEC):
        res[pl.ds(j * 16, 16)] = acc[j]
    pltpu.sync_copy(res, shared.at[pl.ds(s * F, F)])
    plsc.subcore_barrier()

    @pl.when(h == 0)
    def _():
        pltpu.sync_copy(shared.at[pl.ds((s + 1) * F, F)], res2)
        for j in range(NVEC):
            res[pl.ds(j * 16, 16)] = jnp.maximum(
                res[pl.ds(j * 16, 16)], res2[pl.ds(j * 16, 16)]
            )
        pltpu.sync_copy(res, out_hbm.at[pl.ds(b * F, F)])


def _copy_and_head_max(f_ref, x_ref, o_ref, p_ref):
    o_ref[...] = f_ref[...]
    p_ref[0, 0, :] = jnp.max(x_ref[0], axis=0)


_tc_part = pl.pallas_call(
    _copy_and_head_max,
    grid=(BATCH,),
    in_specs=[
        pl.BlockSpec((1, N_NODES, F), lambda i: (i, 0, 0)),
        pl.BlockSpec((1, TC_ROWS, F), lambda i: (i, 0, 0)),
    ],
    out_specs=[
        pl.BlockSpec((1, N_NODES, F), lambda i: (i, 0, 0)),
        pl.BlockSpec((1, 1, F), lambda i: (i, 0, 0)),
    ],
    out_shape=[
        jax.ShapeDtypeStruct((BATCH, N_NODES, F), jnp.float32),
        jax.ShapeDtypeStruct((BATCH, 1, F), jnp.float32),
    ],
)


def kernel(filtres, X):
    sc_pooled = _segment_max_sc(X.reshape(-1))
    filtres_out, head_max = _tc_part(filtres, X)
    pooled = jnp.maximum(sc_pooled.reshape(BATCH, F), head_max.reshape(BATCH, F))
    return (filtres_out, pooled)


# speedup vs baseline: 4.7739x; 1.0026x over previous
import functools

import jax
import jax.numpy as jnp
from jax import lax
from jax.experimental import pallas as pl
from jax.experimental.pallas import tpu as pltpu
from jax.experimental.pallas import tpu_sc as plsc

BATCH = 16
N_NODES = 4096
F = 128
NVEC = F // 16
TC_ROWS = 256
SC_ROWS = N_NODES - TC_ROWS
ROWS_PER_W = SC_ROWS // 2
RCHUNK = 240
NCHUNK = ROWS_PER_W // RCHUNK
UNROLL = 2

_mesh = plsc.VectorSubcoreMesh(core_axis_name="c", subcore_axis_name="s")


@functools.partial(
    pl.kernel,
    out_type=jax.ShapeDtypeStruct((BATCH * F,), jnp.float32),
    mesh=_mesh,
    scratch_types=[
        pltpu.VMEM((RCHUNK * F,), jnp.float32),
        pltpu.VMEM((RCHUNK * F,), jnp.float32),
        pltpu.VMEM((F,), jnp.float32),
        pltpu.VMEM((F,), jnp.float32),
        pltpu.VMEM_SHARED((16 * F,), jnp.float32),
        pltpu.SemaphoreType.DMA,
        pltpu.SemaphoreType.DMA,
    ],
)
def _segment_max_sc(x_hbm, out_hbm, buf0, buf1, res, res2, shared, sem0, sem1):
    c = lax.axis_index("c")
    s = lax.axis_index("s")
    b = c * 8 + s // 2
    h = s % 2
    base = (b * N_NODES + TC_ROWS + h * ROWS_PER_W) * F

    bufs = (buf0, buf1)
    sems = (sem0, sem1)

    def start(i, bsel):
        pltpu.async_copy(
            x_hbm.at[pl.ds(base + i * RCHUNK * F, RCHUNK * F)],
            bufs[bsel],
            sems[bsel],
        )

    def wait(bsel):
        pltpu.make_async_copy(
            x_hbm.at[pl.ds(base, RCHUNK * F)], bufs[bsel], sems[bsel]
        ).wait()

    def fold_chunk(buf, acc):
        def body(it, acc):
            for k in range(UNROLL):
                r = it * UNROLL + k
                acc = tuple(
                    jnp.maximum(acc[j], buf[pl.ds(r * F + j * 16, 16)])
                    for j in range(NVEC)
                )
            return acc
        return lax.fori_loop(0, RCHUNK // UNROLL, body, acc)

    acc = tuple(jnp.full((16,), -jnp.inf, jnp.float32) for _ in range(NVEC))
    start(0, 0)
    start(1, 1)

    def pair_body(i, acc):
        wait(0)
        acc = fold_chunk(buf0, acc)

        @pl.when(2 * i + 2 < NCHUNK)
        def _():
            start(2 * i + 2, 0)

        wait(1)
        acc = fold_chunk(buf1, acc)

        @pl.when(2 * i + 3 < NCHUNK)
        def _():
            start(2 * i + 3, 1)

        return acc

    acc = lax.fori_loop(0, NCHUNK // 2, pair_body, acc)

    for j in range(NVEC):
        res[pl.ds(j * 16, 16)] = acc[j]
    pltpu.sync_copy(res, shared.at[pl.ds(s * F, F)])
    plsc.subcore_barrier()

    @pl.when(h == 0)
    def _():
        pltpu.sync_copy(shared.at[pl.ds((s + 1) * F, F)], res2)
        for j in range(NVEC):
            res[pl.ds(j * 16, 16)] = jnp.maximum(
                res[pl.ds(j * 16, 16)], res2[pl.ds(j * 16, 16)]
            )
        pltpu.sync_copy(res, out_hbm.at[pl.ds(b * F, F)])


def _copy_and_head_max(f_ref, x_ref, o_ref, p_ref):
    o_ref[...] = f_ref[...]
    p_ref[0, 0, :] = jnp.max(x_ref[0], axis=0)


_tc_part = pl.pallas_call(
    _copy_and_head_max,
    grid=(BATCH,),
    in_specs=[
        pl.BlockSpec((1, N_NODES, F), lambda i: (i, 0, 0)),
        pl.BlockSpec((1, TC_ROWS, F), lambda i: (i, 0, 0)),
    ],
    out_specs=[
        pl.BlockSpec((1, N_NODES, F), lambda i: (i, 0, 0)),
        pl.BlockSpec((1, 1, F), lambda i: (i, 0, 0)),
    ],
    out_shape=[
        jax.ShapeDtypeStruct((BATCH, N_NODES, F), jnp.float32),
        jax.ShapeDtypeStruct((BATCH, 1, F), jnp.float32),
    ],
)


def kernel(filtres, X):
    sc_pooled = _segment_max_sc(X.reshape(-1))
    filtres_out, head_max = _tc_part(filtres, X)
    pooled = jnp.maximum(sc_pooled.reshape(BATCH, F), head_max.reshape(BATCH, F))
    return (filtres_out, pooled)
